# Initial kernel scaffold; baseline (speedup 1.0000x reference)
import jax
import jax.numpy as jnp
from jax.experimental import pallas as pl


def kernel(node_emb, rel_emb, basis1, comp1, root1, bias1, basis2, comp2, root2, bias2, edge_index, edge_type, h_idx, r_idx, t_idx):
    raise NotImplementedError("write your pallas kernel here")


# speedup vs baseline: 1.1806x; 1.1806x over previous
import functools

import jax
import jax.numpy as jnp
from jax import lax
from jax.experimental import pallas as pl
from jax.experimental.pallas import tpu as pltpu
from jax.experimental.pallas import tpu_sc as plsc

N_NODES = 10000
NPAD = 10240
NUM_RELS = 16
D = 128
H = 64
E = 320000
T = 16384
MARGIN = 9.0

NUM_TILES = 16
E_PER_TILE = E // NUM_TILES
CHUNK = 80
ITERS = E_PER_TILE // CHUNK
ROWS_PER_TILE = NPAD // NUM_TILES

_MESH = plsc.VectorSubcoreMesh(core_axis_name="c", subcore_axis_name="s")


def _make_sc_aggregate(with_cnt):
  @functools.partial(
      pl.kernel, mesh=_MESH,
      out_type=[jax.ShapeDtypeStruct((2, NPAD, D), jnp.float32),
                jax.ShapeDtypeStruct((NPAD,), jnp.float32)],
      scratch_types=[
          pltpu.VMEM((CHUNK,), jnp.int32),
          pltpu.VMEM((CHUNK,), jnp.int32),
          pltpu.VMEM((CHUNK,), jnp.int32),
          pltpu.VMEM((CHUNK, D), jnp.float32),
          pltpu.VMEM((CHUNK + 16,), jnp.float32),
          pltpu.VMEM((CHUNK,), jnp.float32),
          pltpu.VMEM((16,), jnp.float32),
          pltpu.VMEM_SHARED((NPAD, D), jnp.float32),
          pltpu.VMEM_SHARED((NPAD,), jnp.float32),
          pltpu.SemaphoreType.DMA,
      ],
  )
  def k(x_hbm, src_hbm, dst_hbm, typ_hbm, compcols_hbm, zrows_hbm, zcnt_hbm,
        acc_out, cnt_out,
        src_v, dst_v, typ_v, rows_v, gain_v, ones_v, compcol_v,
        acc_sh, cnt_sh, sem):
    c = lax.axis_index("c")
    s = lax.axis_index("s")
    rsl = pl.ds(s * ROWS_PER_TILE, ROWS_PER_TILE)
    pltpu.sync_copy(zrows_hbm.at[rsl], acc_sh.at[rsl])
    if with_cnt:
      pltpu.sync_copy(zcnt_hbm.at[rsl], cnt_sh.at[rsl])
    pltpu.sync_copy(compcols_hbm.at[c], compcol_v)
    for j in range(CHUNK // 16):
      ones_v[pl.ds(j * 16, 16)] = jnp.ones((16,), jnp.float32)
    plsc.subcore_barrier()

    base = s * E_PER_TILE
    comp_vec = compcol_v[...]

    def body(it, carry):
      off = base + it * CHUNK
      pltpu.sync_copy(src_hbm.at[pl.ds(off, CHUNK)], src_v)
      pltpu.sync_copy(dst_hbm.at[pl.ds(off, CHUNK)], dst_v)
      pltpu.sync_copy(typ_hbm.at[pl.ds(off, CHUNK)], typ_v)
      pltpu.async_copy(x_hbm.at[src_v], rows_v, sem).wait()
      for j in range(CHUNK // 16):
        sl = pl.ds(j * 16, 16)
        t16 = typ_v[sl]
        g16 = jnp.zeros((16,), jnp.float32)
        for r in range(NUM_RELS):
          g16 = jnp.where(t16 == r, comp_vec[r], g16)
        gain_v[sl] = g16

      def scale_body(i, carry2):
        g = gain_v[pl.ds(i, 16)][0]
        for j in range(D // 16):
          sl = pl.ds(j * 16, 16)
          rows_v[i, sl] = rows_v[i, sl] * g
        return carry2

      lax.fori_loop(0, CHUNK, scale_body, 0)
      pltpu.sync_copy(rows_v, acc_sh.at[dst_v], add=True)
      if with_cnt:
        @pl.when(c == 0)
        def _():
          pltpu.sync_copy(ones_v, cnt_sh.at[dst_v], add=True)
      return carry

    lax.fori_loop(0, ITERS, body, 0)
    plsc.subcore_barrier()
    pltpu.sync_copy(acc_sh.at[rsl], acc_out.at[c, rsl])
    if with_cnt:
      @pl.when(c == 0)
      def _():
        pltpu.sync_copy(cnt_sh.at[rsl], cnt_out.at[rsl])

  return k


def _tc_combine(acc0, acc1, x, cnt2d, b0, b1, root, bias2d, relu):
  RB = 512

  def body(a0_ref, a1_ref, x_ref, cnt_ref, b0_ref, b1_ref, root_ref,
           bias_ref, o_ref):
    msg = jnp.dot(a0_ref[...], b0_ref[...], preferred_element_type=jnp.float32)
    msg = msg + jnp.dot(a1_ref[...], b1_ref[...],
                        preferred_element_type=jnp.float32)
    denom = jnp.maximum(cnt_ref[...], 1.0)
    o = msg / denom + jnp.dot(x_ref[...], root_ref[...],
                              preferred_element_type=jnp.float32)
    o = o + bias_ref[...]
    if relu:
      o = jnp.maximum(o, 0.0)
    o_ref[...] = o

  return pl.pallas_call(
      body,
      grid=(NPAD // RB,),
      in_specs=[
          pl.BlockSpec((RB, D), lambda i: (i, 0)),
          pl.BlockSpec((RB, D), lambda i: (i, 0)),
          pl.BlockSpec((RB, D), lambda i: (i, 0)),
          pl.BlockSpec((RB, 1), lambda i: (i, 0)),
          pl.BlockSpec((D, D), lambda i: (0, 0)),
          pl.BlockSpec((D, D), lambda i: (0, 0)),
          pl.BlockSpec((D, D), lambda i: (0, 0)),
          pl.BlockSpec((1, D), lambda i: (0, 0)),
      ],
      out_specs=pl.BlockSpec((RB, D), lambda i: (i, 0)),
      out_shape=jax.ShapeDtypeStruct((NPAD, D), jnp.float32),
  )(acc0, acc1, x, cnt2d, b0, b1, root, bias2d)


NW = 32
T_PER_W = T // NW
TCHUNK = 128
TITERS = T_PER_W // TCHUNK


@functools.partial(
    pl.kernel, mesh=_MESH,
    out_type=[jax.ShapeDtypeStruct((T, D), jnp.float32),
              jax.ShapeDtypeStruct((T, D), jnp.float32),
              jax.ShapeDtypeStruct((T, D), jnp.float32)],
    scratch_types=[
        pltpu.VMEM((TCHUNK,), jnp.int32),
        pltpu.VMEM((TCHUNK,), jnp.int32),
        pltpu.VMEM((TCHUNK,), jnp.int32),
        pltpu.VMEM((TCHUNK, D), jnp.float32),
        pltpu.VMEM((TCHUNK, D), jnp.float32),
        pltpu.VMEM((TCHUNK, D), jnp.float32),
        pltpu.SemaphoreType.DMA,
    ],
)
def _sc_triple_gather(x_hbm, rel_hbm, hidx_hbm, tidx_hbm, ridx_hbm,
                      hrows_out, trows_out, ph_out,
                      hi_v, ti_v, ri_v, hb_v, tb_v, ph_v, sem):
  c = lax.axis_index("c")
  s = lax.axis_index("s")
  wid = s * 2 + c
  base = wid * T_PER_W

  def body(it, carry):
    off = base + it * TCHUNK
    sl = pl.ds(off, TCHUNK)
    pltpu.sync_copy(hidx_hbm.at[sl], hi_v)
    pltpu.sync_copy(tidx_hbm.at[sl], ti_v)
    pltpu.sync_copy(ridx_hbm.at[sl], ri_v)
    pltpu.async_copy(x_hbm.at[hi_v], hb_v, sem).wait()
    pltpu.async_copy(x_hbm.at[ti_v], tb_v, sem).wait()
    pltpu.async_copy(rel_hbm.at[ri_v], ph_v, sem).wait()
    pltpu.sync_copy(hb_v, hrows_out.at[sl])
    pltpu.sync_copy(tb_v, trows_out.at[sl])
    pltpu.sync_copy(ph_v, ph_out.at[sl])
    return carry

  lax.fori_loop(0, TITERS, body, 0)


def _tc_score(hrows, trows, ph):
  RB = 512

  def body(h_ref, t_ref, p_ref, o_ref):
    h = h_ref[...]
    t = t_ref[...]
    p = p_ref[...][:, :H]
    r_re = jnp.cos(p)
    r_im = jnp.sin(p)
    h_re = h[:, :H]
    h_im = h[:, H:]
    s_re = h_re * r_re - h_im * r_im - t[:, :H]
    s_im = h_re * r_im + h_im * r_re - t[:, H:]
    dist = jnp.sqrt(s_re * s_re + s_im * s_im).sum(axis=1, keepdims=True)
    o_ref[...] = MARGIN - dist

  return pl.pallas_call(
      body,
      grid=(T // RB,),
      in_specs=[
          pl.BlockSpec((RB, D), lambda i: (i, 0)),
          pl.BlockSpec((RB, D), lambda i: (i, 0)),
          pl.BlockSpec((RB, D), lambda i: (i, 0)),
      ],
      out_specs=pl.BlockSpec((RB, 1), lambda i: (i, 0)),
      out_shape=jax.ShapeDtypeStruct((T, 1), jnp.float32),
  )(hrows, trows, ph)


_sc_agg_cnt = _make_sc_aggregate(with_cnt=True)
_sc_agg_nocnt = _make_sc_aggregate(with_cnt=False)


def kernel(node_emb, rel_emb, basis1, comp1, root1, bias1,
           basis2, comp2, root2, bias2,
           edge_index, edge_type, h_idx, r_idx, t_idx):
  src = edge_index[0].astype(jnp.int32)
  dst = edge_index[1].astype(jnp.int32)
  typ = edge_type.astype(jnp.int32)
  zrows = jnp.zeros((NPAD, D), jnp.float32)
  zcnt = jnp.zeros((NPAD,), jnp.float32)
  x0 = jnp.zeros((NPAD, D), jnp.float32).at[:N_NODES].set(node_emb)

  acc, cnt = _sc_agg_cnt(x0, src, dst, typ,
                         comp1.T.astype(jnp.float32), zrows, zcnt)
  cnt2d = cnt.reshape(NPAD, 1)
  x1 = _tc_combine(acc[0], acc[1], x0, cnt2d, basis1[0], basis1[1],
                   root1, bias1.reshape(1, D), relu=True)

  acc2, _ = _sc_agg_nocnt(x1, src, dst, typ,
                          comp2.T.astype(jnp.float32), zrows, zcnt)
  x2 = _tc_combine(acc2[0], acc2[1], x1, cnt2d, basis2[0], basis2[1],
                   root2, bias2.reshape(1, D), relu=False)

  rel_pad = jnp.zeros((NUM_RELS, D), jnp.float32).at[:, :H].set(rel_emb)
  hrows, trows, ph = _sc_triple_gather(
      x2, rel_pad, h_idx.astype(jnp.int32), t_idx.astype(jnp.int32),
      r_idx.astype(jnp.int32))
  return _tc_score(hrows, trows, ph).reshape(T)


# speedup vs baseline: 1.6380x; 1.3874x over previous
import functools

import jax
import jax.numpy as jnp
from jax import lax
from jax.experimental import pallas as pl
from jax.experimental.pallas import tpu as pltpu
from jax.experimental.pallas import tpu_sc as plsc

N_NODES = 10000
NPAD = 10240
NUM_RELS = 16
D = 128
H = 64
E = 320000
T = 16384
MARGIN = 9.0

NUM_TILES = 16
CB = 128
NCHUNK = 160
E_PAD = NUM_TILES * NCHUNK * CB
NROWS = E_PAD // CB
E_PER_TILE = NCHUNK * CB
ROWS_PER_TILE = NPAD // NUM_TILES
TBL_HALF = NUM_RELS * NPAD

_MESH = plsc.VectorSubcoreMesh(core_axis_name="c", subcore_axis_name="s")


def _tc_build_scaled(comp_flat, x):
  RB = 512

  def body(comp_ref, x_ref, o_ref):
    g = comp_ref[pl.program_id(0)]
    o_ref[...] = (x_ref[...] * g)[None]

  return pl.pallas_call(
      body,
      grid=(2 * NUM_RELS, NPAD // RB),
      in_specs=[
          pl.BlockSpec(memory_space=pltpu.SMEM),
          pl.BlockSpec((RB, D), lambda i, j: (j, 0)),
      ],
      out_specs=pl.BlockSpec((1, RB, D), lambda i, j: (i, j, 0)),
      out_shape=jax.ShapeDtypeStruct((2 * NUM_RELS, NPAD, D), jnp.float32),
  )(comp_flat, x)


def _tc_edge_indices(src2d, typ2d):
  def body(src_ref, typ_ref, o_ref):
    base = typ_ref[...] * NPAD + src_ref[...]
    o_ref[0] = base
    o_ref[1] = base + TBL_HALF

  return pl.pallas_call(
      body,
      out_shape=jax.ShapeDtypeStruct((2, NROWS, CB), jnp.int32),
  )(src2d, typ2d)


def _make_sc_aggregate(with_cnt):
  @functools.partial(
      pl.kernel, mesh=_MESH,
      out_type=[jax.ShapeDtypeStruct((2, NPAD, D), jnp.float32),
                jax.ShapeDtypeStruct((NPAD,), jnp.float32)],
      scratch_types=[
          pltpu.VMEM((CB,), jnp.int32),
          pltpu.VMEM((CB,), jnp.int32),
          pltpu.VMEM((CB,), jnp.int32),
          pltpu.VMEM((CB,), jnp.int32),
          pltpu.VMEM((CB, D), jnp.float32),
          pltpu.VMEM((CB, D), jnp.float32),
          pltpu.VMEM((CB,), jnp.float32),
          pltpu.VMEM_SHARED((NPAD, D), jnp.float32),
          pltpu.VMEM_SHARED((NPAD,), jnp.float32),
          pltpu.SemaphoreType.DMA,
          pltpu.SemaphoreType.DMA,
      ],
  )
  def k(tbl_hbm, idx_hbm, dst_hbm, zrows_hbm, zcnt_hbm,
        acc_out, cnt_out,
        idxc0, idxc1, dstc0, dstc1,
        rows0, rows1, ones_v, acc_sh, cnt_sh, sem, sem2):
    c = lax.axis_index("c")
    s = lax.axis_index("s")
    rsl = pl.ds(s * ROWS_PER_TILE, ROWS_PER_TILE)
    pltpu.sync_copy(zrows_hbm.at[rsl], acc_sh.at[rsl])
    if with_cnt:
      pltpu.sync_copy(zcnt_hbm.at[rsl], cnt_sh.at[rsl])
    for j in range(CB // 16):
      ones_v[pl.ds(j * 16, 16)] = jnp.ones((16,), jnp.float32)
    plsc.subcore_barrier()

    ebase = s * E_PER_TILE
    ibase = c * E_PAD + ebase

    def idxdma(chunk, idxc, dstc):
      pltpu.async_copy(idx_hbm.at[pl.ds(ibase + chunk * CB, CB)], idxc, sem2)
      pltpu.async_copy(dst_hbm.at[pl.ds(ebase + chunk * CB, CB)], dstc, sem2)

    def idxwait(idxc, dstc):
      pltpu.make_async_copy(idx_hbm.at[pl.ds(0, CB)], idxc, sem2).wait()
      pltpu.make_async_copy(dst_hbm.at[pl.ds(0, CB)], dstc, sem2).wait()

    def gstart(idxc, buf):
      pltpu.async_copy(tbl_hbm.at[idxc], buf, sem)

    def gwait(buf):
      pltpu.make_async_copy(tbl_hbm.at[idxc0], buf, sem).wait()

    def scat(dstc, buf):
      pltpu.sync_copy(buf, acc_sh.at[dstc], add=True)
      if with_cnt:
        @pl.when(c == 0)
        def _():
          pltpu.sync_copy(ones_v, cnt_sh.at[dstc], add=True)

    idxdma(0, idxc0, dstc0)
    idxwait(idxc0, dstc0)
    gstart(idxc0, rows0)

    def body(k2, carry):
      ch = k2 * 2
      idxdma(ch + 1, idxc1, dstc1)
      gwait(rows0)
      idxwait(idxc1, dstc1)
      gstart(idxc1, rows1)
      scat(dstc0, rows0)

      @pl.when(ch + 2 < NCHUNK)
      def _():
        idxdma(ch + 2, idxc0, dstc0)

      gwait(rows1)

      @pl.when(ch + 2 < NCHUNK)
      def _():
        idxwait(idxc0, dstc0)
        gstart(idxc0, rows0)

      scat(dstc1, rows1)
      return carry

    lax.fori_loop(0, NCHUNK // 2, body, 0)

    plsc.subcore_barrier()
    pltpu.sync_copy(acc_sh.at[rsl], acc_out.at[c, rsl])
    if with_cnt:
      @pl.when(c == 0)
      def _():
        pltpu.sync_copy(cnt_sh.at[rsl], cnt_out.at[rsl])

  return k


def _tc_combine(acc0, acc1, x, cnt2d, b0, b1, root, bias2d, relu):
  RB = 512

  def body(a0_ref, a1_ref, x_ref, cnt_ref, b0_ref, b1_ref, root_ref,
           bias_ref, o_ref):
    msg = jnp.dot(a0_ref[...], b0_ref[...], preferred_element_type=jnp.float32)
    msg = msg + jnp.dot(a1_ref[...], b1_ref[...],
                        preferred_element_type=jnp.float32)
    denom = jnp.maximum(cnt_ref[...], 1.0)
    o = msg / denom + jnp.dot(x_ref[...], root_ref[...],
                              preferred_element_type=jnp.float32)
    o = o + bias_ref[...]
    if relu:
      o = jnp.maximum(o, 0.0)
    o_ref[...] = o

  return pl.pallas_call(
      body,
      grid=(NPAD // RB,),
      in_specs=[
          pl.BlockSpec((RB, D), lambda i: (i, 0)),
          pl.BlockSpec((RB, D), lambda i: (i, 0)),
          pl.BlockSpec((RB, D), lambda i: (i, 0)),
          pl.BlockSpec((RB, 1), lambda i: (i, 0)),
          pl.BlockSpec((D, D), lambda i: (0, 0)),
          pl.BlockSpec((D, D), lambda i: (0, 0)),
          pl.BlockSpec((D, D), lambda i: (0, 0)),
          pl.BlockSpec((1, D), lambda i: (0, 0)),
      ],
      out_specs=pl.BlockSpec((RB, D), lambda i: (i, 0)),
      out_shape=jax.ShapeDtypeStruct((NPAD, D), jnp.float32),
  )(acc0, acc1, x, cnt2d, b0, b1, root, bias2d)


NW = 32
T_PER_W = T // NW
TCHUNK = 128
TITERS = T_PER_W // TCHUNK


@functools.partial(
    pl.kernel, mesh=_MESH,
    out_type=[jax.ShapeDtypeStruct((T, D), jnp.float32),
              jax.ShapeDtypeStruct((T, D), jnp.float32),
              jax.ShapeDtypeStruct((T, D), jnp.float32)],
    scratch_types=[
        pltpu.VMEM((TCHUNK,), jnp.int32),
        pltpu.VMEM((TCHUNK,), jnp.int32),
        pltpu.VMEM((TCHUNK,), jnp.int32),
        pltpu.VMEM((TCHUNK, D), jnp.float32),
        pltpu.VMEM((TCHUNK, D), jnp.float32),
        pltpu.VMEM((TCHUNK, D), jnp.float32),
        pltpu.SemaphoreType.DMA,
    ],
)
def _sc_triple_gather(x_hbm, rel_hbm, hidx_hbm, tidx_hbm, ridx_hbm,
                      hrows_out, trows_out, ph_out,
                      hi_v, ti_v, ri_v, hb_v, tb_v, ph_v, sem):
  c = lax.axis_index("c")
  s = lax.axis_index("s")
  wid = s * 2 + c
  base = wid * T_PER_W

  def body(it, carry):
    off = base + it * TCHUNK
    sl = pl.ds(off, TCHUNK)
    pltpu.sync_copy(hidx_hbm.at[sl], hi_v)
    pltpu.sync_copy(tidx_hbm.at[sl], ti_v)
    pltpu.sync_copy(ridx_hbm.at[sl], ri_v)
    pltpu.async_copy(x_hbm.at[hi_v], hb_v, sem).wait()
    pltpu.async_copy(x_hbm.at[ti_v], tb_v, sem).wait()
    pltpu.async_copy(rel_hbm.at[ri_v], ph_v, sem).wait()
    pltpu.sync_copy(hb_v, hrows_out.at[sl])
    pltpu.sync_copy(tb_v, trows_out.at[sl])
    pltpu.sync_copy(ph_v, ph_out.at[sl])
    return carry

  lax.fori_loop(0, TITERS, body, 0)


def _tc_score(hrows, trows, ph):
  RB = 512

  def body(h_ref, t_ref, p_ref, o_ref):
    h = h_ref[...]
    t = t_ref[...]
    p = p_ref[...][:, :H]
    r_re = jnp.cos(p)
    r_im = jnp.sin(p)
    h_re = h[:, :H]
    h_im = h[:, H:]
    s_re = h_re * r_re - h_im * r_im - t[:, :H]
    s_im = h_re * r_im + h_im * r_re - t[:, H:]
    dist = jnp.sqrt(s_re * s_re + s_im * s_im).sum(axis=1, keepdims=True)
    o_ref[...] = MARGIN - dist

  return pl.pallas_call(
      body,
      grid=(T // RB,),
      in_specs=[
          pl.BlockSpec((RB, D), lambda i: (i, 0)),
          pl.BlockSpec((RB, D), lambda i: (i, 0)),
          pl.BlockSpec((RB, D), lambda i: (i, 0)),
      ],
      out_specs=pl.BlockSpec((RB, 1), lambda i: (i, 0)),
      out_shape=jax.ShapeDtypeStruct((T, 1), jnp.float32),
  )(hrows, trows, ph)


_sc_agg_cnt = _make_sc_aggregate(with_cnt=True)
_sc_agg_nocnt = _make_sc_aggregate(with_cnt=False)


def kernel(node_emb, rel_emb, basis1, comp1, root1, bias1,
           basis2, comp2, root2, bias2,
           edge_index, edge_type, h_idx, r_idx, t_idx):
  src = edge_index[0].astype(jnp.int32)
  dst = edge_index[1].astype(jnp.int32)
  typ = edge_type.astype(jnp.int32)
  npad_e = E_PAD - E
  pad_ar = jnp.arange(npad_e, dtype=jnp.int32)
  pad_dst = N_NODES + pad_ar % (NPAD - N_NODES)
  pad_src = pad_ar % N_NODES
  src2d = jnp.concatenate([src, pad_src]).reshape(NROWS, CB)
  dst1d = jnp.concatenate([dst, pad_dst])
  typ2d = jnp.concatenate(
      [typ, jnp.zeros((npad_e,), jnp.int32)]).reshape(NROWS, CB)

  idx1d = _tc_edge_indices(src2d, typ2d).reshape(2 * E_PAD)
  zrows = jnp.zeros((NPAD, D), jnp.float32)
  zcnt = jnp.zeros((NPAD,), jnp.float32)
  x0 = jnp.zeros((NPAD, D), jnp.float32).at[:N_NODES].set(node_emb)

  scaled1 = _tc_build_scaled(
      comp1.T.reshape(2 * NUM_RELS).astype(jnp.float32), x0
  ).reshape(2 * TBL_HALF, D)
  acc, cnt = _sc_agg_cnt(scaled1, idx1d, dst1d, zrows, zcnt)
  cnt2d = cnt.reshape(NPAD, 1)
  x1 = _tc_combine(acc[0], acc[1], x0, cnt2d, basis1[0], basis1[1],
                   root1, bias1.reshape(1, D), relu=True)

  scaled2 = _tc_build_scaled(
      comp2.T.reshape(2 * NUM_RELS).astype(jnp.float32), x1
  ).reshape(2 * TBL_HALF, D)
  acc2, _ = _sc_agg_nocnt(scaled2, idx1d, dst1d, zrows, zcnt)
  x2 = _tc_combine(acc2[0], acc2[1], x1, cnt2d, basis2[0], basis2[1],
                   root2, bias2.reshape(1, D), relu=False)

  rel_pad = jnp.zeros((NUM_RELS, D), jnp.float32).at[:, :H].set(rel_emb)
  hrows, trows, ph = _sc_triple_gather(
      x2, rel_pad, h_idx.astype(jnp.int32), t_idx.astype(jnp.int32),
      r_idx.astype(jnp.int32))
  return _tc_score(hrows, trows, ph).reshape(T)


# speedup vs baseline: 2.2884x; 1.3970x over previous
import functools

import jax
import jax.numpy as jnp
from jax import lax
from jax.experimental import pallas as pl
from jax.experimental.pallas import tpu as pltpu
from jax.experimental.pallas import tpu_sc as plsc

N_NODES = 10000
NPAD = 10240
NUM_RELS = 16
D = 128
H = 64
E = 320000
T = 16384
MARGIN = 9.0

NUM_TILES = 16
CB = 64
NCHUNK = 320
E_PAD = NUM_TILES * NCHUNK * CB
NROWS = E_PAD // CB
E_PER_TILE = NCHUNK * CB
ROWS_PER_TILE = NPAD // NUM_TILES
TBL_HALF = NUM_RELS * NPAD

_MESH = plsc.VectorSubcoreMesh(core_axis_name="c", subcore_axis_name="s")


def _tc_build_scaled(comp_flat, x):
  RB = 512

  def body(comp_ref, x_ref, o_ref):
    g = comp_ref[pl.program_id(0)]
    o_ref[...] = (x_ref[...] * g)[None]

  return pl.pallas_call(
      body,
      grid=(2 * NUM_RELS, NPAD // RB),
      in_specs=[
          pl.BlockSpec(memory_space=pltpu.SMEM),
          pl.BlockSpec((RB, D), lambda i, j: (j, 0)),
      ],
      out_specs=pl.BlockSpec((1, RB, D), lambda i, j: (i, j, 0)),
      out_shape=jax.ShapeDtypeStruct((2 * NUM_RELS, NPAD, D), jnp.float32),
  )(comp_flat, x)


def _tc_edge_indices(src2d, typ2d):
  def body(src_ref, typ_ref, o_ref):
    base = typ_ref[...] * NPAD + src_ref[...]
    o_ref[0] = base
    o_ref[1] = base + TBL_HALF

  return pl.pallas_call(
      body,
      out_shape=jax.ShapeDtypeStruct((2, NROWS, CB), jnp.int32),
  )(src2d, typ2d)


def _make_sc_aggregate(with_cnt):
  @functools.partial(
      pl.kernel, mesh=_MESH,
      out_type=[jax.ShapeDtypeStruct((2, NPAD, D), jnp.float32),
                jax.ShapeDtypeStruct((NPAD,), jnp.float32)],
      scratch_types=[
          pltpu.VMEM((CB,), jnp.int32),
          pltpu.VMEM((CB,), jnp.int32),
          pltpu.VMEM((CB,), jnp.int32),
          pltpu.VMEM((CB,), jnp.int32),
          pltpu.VMEM((CB,), jnp.int32),
          pltpu.VMEM((CB,), jnp.int32),
          pltpu.VMEM((CB,), jnp.int32),
          pltpu.VMEM((CB,), jnp.int32),
          pltpu.VMEM((CB, D), jnp.float32),
          pltpu.VMEM((CB, D), jnp.float32),
          pltpu.VMEM((CB, D), jnp.float32),
          pltpu.VMEM((CB, D), jnp.float32),
          pltpu.VMEM((CB,), jnp.float32),
          pltpu.VMEM_SHARED((NPAD, D), jnp.float32),
          pltpu.VMEM_SHARED((NPAD,), jnp.float32),
          pltpu.SemaphoreType.DMA,
          pltpu.SemaphoreType.DMA,
          pltpu.SemaphoreType.DMA,
          pltpu.SemaphoreType.DMA,
          pltpu.SemaphoreType.DMA,
          pltpu.SemaphoreType.DMA,
          pltpu.SemaphoreType.DMA,
      ],
  )
  def k(tbl_hbm, idx_hbm, dst_hbm, zrows_hbm, zcnt_hbm,
        acc_out, cnt_out,
        idxs0, idxs1, idxs2, idxs3, dstc0, dstc1, dstc2, dstc3,
        rows0, rows1, rows2, rows3, ones_v, acc_sh, cnt_sh,
        gsem0, gsem1, sem2, ssem0, ssem1, csem0, csem1):
    c = lax.axis_index("c")
    s = lax.axis_index("s")
    rsl = pl.ds(s * ROWS_PER_TILE, ROWS_PER_TILE)
    pltpu.sync_copy(zrows_hbm.at[rsl], acc_sh.at[rsl])
    if with_cnt:
      pltpu.sync_copy(zcnt_hbm.at[rsl], cnt_sh.at[rsl])
    for j in range(CB // 16):
      ones_v[pl.ds(j * 16, 16)] = jnp.ones((16,), jnp.float32)
    plsc.subcore_barrier()

    ebase = s * E_PER_TILE
    ibase = c * E_PAD + ebase
    rows = [rows0, rows1, rows2, rows3]
    dstc = [dstc0, dstc1, dstc2, dstc3]
    idxs = [idxs0, idxs1, idxs2, idxs3]
    gsem = [gsem0, gsem1]
    ssem = [ssem0, ssem1]
    csem = [csem0, csem1]

    def idxdma(chunk, j):
      pltpu.async_copy(idx_hbm.at[pl.ds(ibase + chunk * CB, CB)],
                       idxs[j], sem2)

    def dstdma(chunk, j):
      pltpu.async_copy(dst_hbm.at[pl.ds(ebase + chunk * CB, CB)],
                       dstc[j], sem2)

    def fdrain(n):
      for _ in range(n):
        pltpu.make_async_copy(dst_hbm.at[pl.ds(0, CB)], dstc0, sem2).wait()

    def gstart(j):
      pltpu.async_copy(tbl_hbm.at[idxs[j]], rows[j], gsem[j % 2])

    def gwait(j):
      pltpu.make_async_copy(tbl_hbm.at[idxs[0]], rows[j],
                            gsem[j % 2]).wait()

    def sstart(j):
      pltpu.async_copy(rows[j], acc_sh.at[dstc[j]], ssem[j % 2], add=True)
      if with_cnt:
        @pl.when(c == 0)
        def _():
          pltpu.async_copy(ones_v, cnt_sh.at[dstc[j]], csem[j % 2], add=True)

    def swait(j):
      pltpu.make_async_copy(rows0, acc_sh.at[dstc0], ssem[j % 2]).wait()
      if with_cnt:
        @pl.when(c == 0)
        def _():
          pltpu.make_async_copy(ones_v, cnt_sh.at[dstc0], csem[j % 2]).wait()

    def slot(q, j, do_swait=True, drain_n=2, g2=True, d2=True, i3=True):
      if do_swait:
        swait(j)
      fdrain(drain_n)
      gwait(j)
      sstart(j)
      if g2:
        gstart((j + 2) % 4)
      if d2:
        dstdma(q + 2, (j + 2) % 4)
      if i3:
        idxdma(q + 3, (j + 3) % 4)

    dstdma(0, 0)
    idxdma(0, 0)
    dstdma(1, 1)
    idxdma(1, 1)
    fdrain(4)
    gstart(0)
    gstart(1)
    idxdma(2, 2)
    slot(0, 0, do_swait=False, drain_n=1)
    slot(1, 1, do_swait=False)
    slot(2, 2)
    slot(3, 3)

    def body(it, carry):
      q0 = it * 4
      slot(q0 + 0, 0)
      slot(q0 + 1, 1)
      slot(q0 + 2, 2)
      slot(q0 + 3, 3)
      return carry

    lax.fori_loop(1, NCHUNK // 4 - 1, body, 0)
    q0 = NCHUNK - 4
    slot(q0 + 0, 0)
    slot(q0 + 1, 1, i3=False)
    slot(q0 + 2, 2, drain_n=1, g2=False, d2=False, i3=False)
    slot(q0 + 3, 3, drain_n=0, g2=False, d2=False, i3=False)
    swait(0)
    swait(1)

    plsc.subcore_barrier()
    pltpu.sync_copy(acc_sh.at[rsl], acc_out.at[c, rsl])
    if with_cnt:
      @pl.when(c == 0)
      def _():
        pltpu.sync_copy(cnt_sh.at[rsl], cnt_out.at[rsl])

  return k


def _tc_combine(acc0, acc1, x, cnt2d, b0, b1, root, bias2d, relu,
                next_comp=None):
  RB = 512

  def compute(a0_ref, a1_ref, x_ref, cnt_ref, b0_ref, b1_ref, root_ref,
              bias_ref):
    msg = jnp.dot(a0_ref[...], b0_ref[...], preferred_element_type=jnp.float32)
    msg = msg + jnp.dot(a1_ref[...], b1_ref[...],
                        preferred_element_type=jnp.float32)
    denom = jnp.maximum(cnt_ref[...], 1.0)
    o = msg / denom + jnp.dot(x_ref[...], root_ref[...],
                              preferred_element_type=jnp.float32)
    o = o + bias_ref[...]
    if relu:
      o = jnp.maximum(o, 0.0)
    return o

  row_specs = [
      pl.BlockSpec((RB, D), lambda i: (i, 0)),
      pl.BlockSpec((RB, D), lambda i: (i, 0)),
      pl.BlockSpec((RB, D), lambda i: (i, 0)),
      pl.BlockSpec((RB, 1), lambda i: (i, 0)),
      pl.BlockSpec((D, D), lambda i: (0, 0)),
      pl.BlockSpec((D, D), lambda i: (0, 0)),
      pl.BlockSpec((D, D), lambda i: (0, 0)),
      pl.BlockSpec((1, D), lambda i: (0, 0)),
  ]

  if next_comp is None:
    def body(a0_ref, a1_ref, x_ref, cnt_ref, b0_ref, b1_ref, root_ref,
             bias_ref, o_ref):
      o_ref[...] = compute(a0_ref, a1_ref, x_ref, cnt_ref, b0_ref, b1_ref,
                           root_ref, bias_ref)

    return pl.pallas_call(
        body,
        grid=(NPAD // RB,),
        in_specs=row_specs,
        out_specs=pl.BlockSpec((RB, D), lambda i: (i, 0)),
        out_shape=jax.ShapeDtypeStruct((NPAD, D), jnp.float32),
    )(acc0, acc1, x, cnt2d, b0, b1, root, bias2d)

  def body2(comp_ref, a0_ref, a1_ref, x_ref, cnt_ref, b0_ref, b1_ref,
            root_ref, bias_ref, o_ref, t_ref):
    o = compute(a0_ref, a1_ref, x_ref, cnt_ref, b0_ref, b1_ref,
                root_ref, bias_ref)
    o_ref[...] = o
    for kk in range(2 * NUM_RELS):
      t_ref[kk] = o * comp_ref[kk]

  return pl.pallas_call(
      body2,
      grid=(NPAD // RB,),
      in_specs=[pl.BlockSpec(memory_space=pltpu.SMEM)] + row_specs,
      out_specs=[
          pl.BlockSpec((RB, D), lambda i: (i, 0)),
          pl.BlockSpec((2 * NUM_RELS, RB, D), lambda i: (0, i, 0)),
      ],
      out_shape=[
          jax.ShapeDtypeStruct((NPAD, D), jnp.float32),
          jax.ShapeDtypeStruct((2 * NUM_RELS, NPAD, D), jnp.float32),
      ],
  )(next_comp, acc0, acc1, x, cnt2d, b0, b1, root, bias2d)


NW = 32
T_PER_W = T // NW
TCHUNK = 128
TITERS = T_PER_W // TCHUNK


@functools.partial(
    pl.kernel, mesh=_MESH,
    out_type=[jax.ShapeDtypeStruct((T, D), jnp.float32),
              jax.ShapeDtypeStruct((T, D), jnp.float32),
              jax.ShapeDtypeStruct((T, D), jnp.float32)],
    scratch_types=[
        pltpu.VMEM((TCHUNK,), jnp.int32),
        pltpu.VMEM((TCHUNK,), jnp.int32),
        pltpu.VMEM((TCHUNK,), jnp.int32),
        pltpu.VMEM((TCHUNK, D), jnp.float32),
        pltpu.VMEM((TCHUNK, D), jnp.float32),
        pltpu.VMEM((TCHUNK, D), jnp.float32),
        pltpu.SemaphoreType.DMA,
    ],
)
def _sc_triple_gather(x_hbm, rel_hbm, hidx_hbm, tidx_hbm, ridx_hbm,
                      hrows_out, trows_out, ph_out,
                      hi_v, ti_v, ri_v, hb_v, tb_v, ph_v, sem):
  c = lax.axis_index("c")
  s = lax.axis_index("s")
  wid = s * 2 + c
  base = wid * T_PER_W

  def body(it, carry):
    off = base + it * TCHUNK
    sl = pl.ds(off, TCHUNK)
    pltpu.sync_copy(hidx_hbm.at[sl], hi_v)
    pltpu.sync_copy(tidx_hbm.at[sl], ti_v)
    pltpu.sync_copy(ridx_hbm.at[sl], ri_v)
    pltpu.async_copy(x_hbm.at[hi_v], hb_v, sem).wait()
    pltpu.async_copy(x_hbm.at[ti_v], tb_v, sem).wait()
    pltpu.async_copy(rel_hbm.at[ri_v], ph_v, sem).wait()
    pltpu.sync_copy(hb_v, hrows_out.at[sl])
    pltpu.sync_copy(tb_v, trows_out.at[sl])
    pltpu.sync_copy(ph_v, ph_out.at[sl])
    return carry

  lax.fori_loop(0, TITERS, body, 0)


def _tc_score(hrows, trows, ph):
  RB = 512

  def body(h_ref, t_ref, p_ref, o_ref):
    h = h_ref[...]
    t = t_ref[...]
    p = p_ref[...][:, :H]
    r_re = jnp.cos(p)
    r_im = jnp.sin(p)
    h_re = h[:, :H]
    h_im = h[:, H:]
    s_re = h_re * r_re - h_im * r_im - t[:, :H]
    s_im = h_re * r_im + h_im * r_re - t[:, H:]
    dist = jnp.sqrt(s_re * s_re + s_im * s_im).sum(axis=1, keepdims=True)
    o_ref[...] = MARGIN - dist

  return pl.pallas_call(
      body,
      grid=(T // RB,),
      in_specs=[
          pl.BlockSpec((RB, D), lambda i: (i, 0)),
          pl.BlockSpec((RB, D), lambda i: (i, 0)),
          pl.BlockSpec((RB, D), lambda i: (i, 0)),
      ],
      out_specs=pl.BlockSpec((RB, 1), lambda i: (i, 0)),
      out_shape=jax.ShapeDtypeStruct((T, 1), jnp.float32),
  )(hrows, trows, ph)


_sc_agg_cnt = _make_sc_aggregate(with_cnt=True)
_sc_agg_nocnt = _make_sc_aggregate(with_cnt=False)


def kernel(node_emb, rel_emb, basis1, comp1, root1, bias1,
           basis2, comp2, root2, bias2,
           edge_index, edge_type, h_idx, r_idx, t_idx):
  src = edge_index[0].astype(jnp.int32)
  dst = edge_index[1].astype(jnp.int32)
  typ = edge_type.astype(jnp.int32)
  npad_e = E_PAD - E
  pad_ar = jnp.arange(npad_e, dtype=jnp.int32)
  pad_dst = N_NODES + pad_ar % (NPAD - N_NODES)
  pad_src = pad_ar % N_NODES
  src2d = jnp.concatenate([src, pad_src]).reshape(NROWS, CB)
  dst1d = jnp.concatenate([dst, pad_dst])
  typ2d = jnp.concatenate(
      [typ, jnp.zeros((npad_e,), jnp.int32)]).reshape(NROWS, CB)

  idx1d = _tc_edge_indices(src2d, typ2d).reshape(2 * E_PAD)
  zrows = jnp.zeros((NPAD, D), jnp.float32)
  zcnt = jnp.zeros((NPAD,), jnp.float32)
  x0 = jnp.zeros((NPAD, D), jnp.float32).at[:N_NODES].set(node_emb)

  scaled1 = _tc_build_scaled(
      comp1.T.reshape(2 * NUM_RELS).astype(jnp.float32), x0
  ).reshape(2 * TBL_HALF, D)
  acc, cnt = _sc_agg_cnt(scaled1, idx1d, dst1d, zrows, zcnt)
  cnt2d = cnt.reshape(NPAD, 1)
  x1, scaled2 = _tc_combine(
      acc[0], acc[1], x0, cnt2d, basis1[0], basis1[1],
      root1, bias1.reshape(1, D), relu=True,
      next_comp=comp2.T.reshape(2 * NUM_RELS).astype(jnp.float32))

  acc2, _ = _sc_agg_nocnt(scaled2.reshape(2 * TBL_HALF, D),
                          idx1d, dst1d, zrows, zcnt)
  x2 = _tc_combine(acc2[0], acc2[1], x1, cnt2d, basis2[0], basis2[1],
                   root2, bias2.reshape(1, D), relu=False)

  rel_pad = jnp.zeros((NUM_RELS, D), jnp.float32).at[:, :H].set(rel_emb)
  hrows, trows, ph = _sc_triple_gather(
      x2, rel_pad, h_idx.astype(jnp.int32), t_idx.astype(jnp.int32),
      r_idx.astype(jnp.int32))
  return _tc_score(hrows, trows, ph).reshape(T)


# speedup vs baseline: 3.4171x; 1.4932x over previous
import functools

import jax
import jax.numpy as jnp
from jax import lax
from jax.experimental import pallas as pl
from jax.experimental.pallas import tpu as pltpu
from jax.experimental.pallas import tpu_sc as plsc

N_NODES = 10000
NPAD = 10240
NUM_RELS = 16
D = 128
H = 64
E = 320000
T = 16384
MARGIN = 9.0

NUM_TILES = 16
CB = 64
NCHUNK = 320
E_PAD = NUM_TILES * NCHUNK * CB
NROWS = E_PAD // CB
E_PER_TILE = NCHUNK * CB
ROWS_PER_TILE = NPAD // NUM_TILES
TBL_HALF = NUM_RELS * NPAD

_MESH = plsc.VectorSubcoreMesh(core_axis_name="c", subcore_axis_name="s")


def _tc_build_scaled(comp_flat, x):
  RB = 512

  def body(comp_ref, x_ref, o_ref):
    xb = x_ref[...]
    for kk in range(2 * NUM_RELS):
      o_ref[kk] = xb * comp_ref[kk]

  return pl.pallas_call(
      body,
      grid=(NPAD // RB,),
      in_specs=[
          pl.BlockSpec(memory_space=pltpu.SMEM),
          pl.BlockSpec((RB, D), lambda i: (i, 0)),
      ],
      out_specs=pl.BlockSpec((2 * NUM_RELS, RB, D), lambda i: (0, i, 0)),
      out_shape=jax.ShapeDtypeStruct((2 * NUM_RELS, NPAD, D), jnp.float32),
  )(comp_flat, x)


def _tc_edge_indices(src2d, typ2d, rel_emb):
  def body(src_ref, typ_ref, rel_ref, o_ref, cs_ref):
    base = typ_ref[...] * NPAD + src_ref[...]
    o_ref[0] = base
    o_ref[1] = base + TBL_HALF
    ph = rel_ref[...]
    cs_ref[...] = jnp.concatenate([jnp.cos(ph), jnp.sin(ph)], axis=1)

  return pl.pallas_call(
      body,
      out_shape=[jax.ShapeDtypeStruct((2, NROWS, CB), jnp.int32),
                 jax.ShapeDtypeStruct((NUM_RELS, D), jnp.float32)],
  )(src2d, typ2d, rel_emb)


def _make_sc_aggregate(with_cnt):
  @functools.partial(
      pl.kernel, mesh=_MESH,
      out_type=[jax.ShapeDtypeStruct((2, NPAD, D), jnp.float32),
                jax.ShapeDtypeStruct((NPAD,), jnp.float32)],
      scratch_types=[
          pltpu.VMEM((CB,), jnp.int32),
          pltpu.VMEM((CB,), jnp.int32),
          pltpu.VMEM((CB,), jnp.int32),
          pltpu.VMEM((CB,), jnp.int32),
          pltpu.VMEM((CB,), jnp.int32),
          pltpu.VMEM((CB,), jnp.int32),
          pltpu.VMEM((CB,), jnp.int32),
          pltpu.VMEM((CB,), jnp.int32),
          pltpu.VMEM((CB, D), jnp.float32),
          pltpu.VMEM((CB, D), jnp.float32),
          pltpu.VMEM((CB, D), jnp.float32),
          pltpu.VMEM((CB, D), jnp.float32),
          pltpu.VMEM((CB,), jnp.float32),
          pltpu.VMEM_SHARED((NPAD, D), jnp.float32),
          pltpu.VMEM_SHARED((NPAD,), jnp.float32),
          pltpu.SemaphoreType.DMA,
          pltpu.SemaphoreType.DMA,
          pltpu.SemaphoreType.DMA,
          pltpu.SemaphoreType.DMA,
          pltpu.SemaphoreType.DMA,
          pltpu.SemaphoreType.DMA,
          pltpu.SemaphoreType.DMA,
      ],
  )
  def k(tbl_hbm, idx_hbm, dst_hbm, zrows_hbm, zcnt_hbm,
        acc_out, cnt_out,
        idxs0, idxs1, idxs2, idxs3, dstc0, dstc1, dstc2, dstc3,
        rows0, rows1, rows2, rows3, ones_v, acc_sh, cnt_sh,
        gsem0, gsem1, sem2, ssem0, ssem1, csem0, csem1):
    c = lax.axis_index("c")
    s = lax.axis_index("s")
    rsl = pl.ds(s * ROWS_PER_TILE, ROWS_PER_TILE)
    pltpu.sync_copy(zrows_hbm.at[rsl], acc_sh.at[rsl])
    if with_cnt:
      pltpu.sync_copy(zcnt_hbm.at[rsl], cnt_sh.at[rsl])
    for j in range(CB // 16):
      ones_v[pl.ds(j * 16, 16)] = jnp.ones((16,), jnp.float32)
    plsc.subcore_barrier()

    ebase = s * E_PER_TILE
    ibase = c * E_PAD + ebase
    rows = [rows0, rows1, rows2, rows3]
    dstc = [dstc0, dstc1, dstc2, dstc3]
    idxs = [idxs0, idxs1, idxs2, idxs3]
    gsem = [gsem0, gsem1]
    ssem = [ssem0, ssem1]
    csem = [csem0, csem1]

    def idxdma(chunk, j):
      pltpu.async_copy(idx_hbm.at[pl.ds(ibase + chunk * CB, CB)],
                       idxs[j], sem2)

    def dstdma(chunk, j):
      pltpu.async_copy(dst_hbm.at[pl.ds(ebase + chunk * CB, CB)],
                       dstc[j], sem2)

    def fdrain(n):
      for _ in range(n):
        pltpu.make_async_copy(dst_hbm.at[pl.ds(0, CB)], dstc0, sem2).wait()

    def gstart(j):
      pltpu.async_copy(tbl_hbm.at[idxs[j]], rows[j], gsem[j % 2])

    def gwait(j):
      pltpu.make_async_copy(tbl_hbm.at[idxs[0]], rows[j],
                            gsem[j % 2]).wait()

    def sstart(j):
      pltpu.async_copy(rows[j], acc_sh.at[dstc[j]], ssem[j % 2], add=True)
      if with_cnt:
        @pl.when(c == 0)
        def _():
          pltpu.async_copy(ones_v, cnt_sh.at[dstc[j]], csem[j % 2], add=True)

    def swait(j):
      pltpu.make_async_copy(rows0, acc_sh.at[dstc0], ssem[j % 2]).wait()
      if with_cnt:
        @pl.when(c == 0)
        def _():
          pltpu.make_async_copy(ones_v, cnt_sh.at[dstc0], csem[j % 2]).wait()

    def slot(q, j, do_swait=True, drain_n=2, g2=True, d2=True, i3=True):
      if do_swait:
        swait(j)
      fdrain(drain_n)
      gwait(j)
      sstart(j)
      if g2:
        gstart((j + 2) % 4)
      if d2:
        dstdma(q + 2, (j + 2) % 4)
      if i3:
        idxdma(q + 3, (j + 3) % 4)

    dstdma(0, 0)
    idxdma(0, 0)
    dstdma(1, 1)
    idxdma(1, 1)
    fdrain(4)
    gstart(0)
    gstart(1)
    idxdma(2, 2)
    slot(0, 0, do_swait=False, drain_n=1)
    slot(1, 1, do_swait=False)
    slot(2, 2)
    slot(3, 3)

    def body(it, carry):
      q0 = it * 4
      slot(q0 + 0, 0)
      slot(q0 + 1, 1)
      slot(q0 + 2, 2)
      slot(q0 + 3, 3)
      return carry

    lax.fori_loop(1, NCHUNK // 4 - 1, body, 0)
    q0 = NCHUNK - 4
    slot(q0 + 0, 0)
    slot(q0 + 1, 1, i3=False)
    slot(q0 + 2, 2, drain_n=1, g2=False, d2=False, i3=False)
    slot(q0 + 3, 3, drain_n=0, g2=False, d2=False, i3=False)
    swait(0)
    swait(1)

    plsc.subcore_barrier()
    pltpu.sync_copy(acc_sh.at[rsl], acc_out.at[c, rsl])
    if with_cnt:
      @pl.when(c == 0)
      def _():
        pltpu.sync_copy(cnt_sh.at[rsl], cnt_out.at[rsl])

  return k


def _tc_combine(acc0, acc1, x, cnt2d, b0, b1, root, bias2d, relu,
                next_comp=None):
  RB = 512

  def compute(a0_ref, a1_ref, x_ref, cnt_ref, b0_ref, b1_ref, root_ref,
              bias_ref):
    msg = jnp.dot(a0_ref[...], b0_ref[...], preferred_element_type=jnp.float32)
    msg = msg + jnp.dot(a1_ref[...], b1_ref[...],
                        preferred_element_type=jnp.float32)
    denom = jnp.maximum(cnt_ref[...], 1.0)
    o = msg / denom + jnp.dot(x_ref[...], root_ref[...],
                              preferred_element_type=jnp.float32)
    o = o + bias_ref[...]
    if relu:
      o = jnp.maximum(o, 0.0)
    return o

  row_specs = [
      pl.BlockSpec((RB, D), lambda i: (i, 0)),
      pl.BlockSpec((RB, D), lambda i: (i, 0)),
      pl.BlockSpec((RB, D), lambda i: (i, 0)),
      pl.BlockSpec((RB, 1), lambda i: (i, 0)),
      pl.BlockSpec((D, D), lambda i: (0, 0)),
      pl.BlockSpec((D, D), lambda i: (0, 0)),
      pl.BlockSpec((D, D), lambda i: (0, 0)),
      pl.BlockSpec((1, D), lambda i: (0, 0)),
  ]

  if next_comp is None:
    def body(a0_ref, a1_ref, x_ref, cnt_ref, b0_ref, b1_ref, root_ref,
             bias_ref, o_ref):
      o_ref[...] = compute(a0_ref, a1_ref, x_ref, cnt_ref, b0_ref, b1_ref,
                           root_ref, bias_ref)

    return pl.pallas_call(
        body,
        grid=(NPAD // RB,),
        in_specs=row_specs,
        out_specs=pl.BlockSpec((RB, D), lambda i: (i, 0)),
        out_shape=jax.ShapeDtypeStruct((NPAD, D), jnp.float32),
    )(acc0, acc1, x, cnt2d, b0, b1, root, bias2d)

  def body2(comp_ref, a0_ref, a1_ref, x_ref, cnt_ref, b0_ref, b1_ref,
            root_ref, bias_ref, o_ref, t_ref):
    o = compute(a0_ref, a1_ref, x_ref, cnt_ref, b0_ref, b1_ref,
                root_ref, bias_ref)
    o_ref[...] = o
    for kk in range(2 * NUM_RELS):
      t_ref[kk] = o * comp_ref[kk]

  return pl.pallas_call(
      body2,
      grid=(NPAD // RB,),
      in_specs=[pl.BlockSpec(memory_space=pltpu.SMEM)] + row_specs,
      out_specs=[
          pl.BlockSpec((RB, D), lambda i: (i, 0)),
          pl.BlockSpec((2 * NUM_RELS, RB, D), lambda i: (0, i, 0)),
      ],
      out_shape=[
          jax.ShapeDtypeStruct((NPAD, D), jnp.float32),
          jax.ShapeDtypeStruct((2 * NUM_RELS, NPAD, D), jnp.float32),
      ],
  )(next_comp, acc0, acc1, x, cnt2d, b0, b1, root, bias2d)


NW = 32
T_PER_W = T // NW
TCHUNK = 128
TITERS = T_PER_W // TCHUNK


@functools.partial(
    pl.kernel, mesh=_MESH,
    out_type=[jax.ShapeDtypeStruct((T, D), jnp.float32),
              jax.ShapeDtypeStruct((T, D), jnp.float32),
              jax.ShapeDtypeStruct((T, D), jnp.float32)],
    scratch_types=[
        pltpu.VMEM((TCHUNK,), jnp.int32),
        pltpu.VMEM((TCHUNK,), jnp.int32),
        pltpu.VMEM((TCHUNK,), jnp.int32),
        pltpu.VMEM((TCHUNK, D), jnp.float32),
        pltpu.VMEM((TCHUNK, D), jnp.float32),
        pltpu.VMEM((TCHUNK, D), jnp.float32),
        pltpu.SemaphoreType.DMA,
    ],
)
def _sc_triple_gather(x_hbm, rel_hbm, hidx_hbm, tidx_hbm, ridx_hbm,
                      hrows_out, trows_out, ph_out,
                      hi_v, ti_v, ri_v, hb_v, tb_v, ph_v, sem):
  c = lax.axis_index("c")
  s = lax.axis_index("s")
  wid = s * 2 + c
  base = wid * T_PER_W

  def body(it, carry):
    off = base + it * TCHUNK
    sl = pl.ds(off, TCHUNK)
    pltpu.sync_copy(hidx_hbm.at[sl], hi_v)
    pltpu.sync_copy(tidx_hbm.at[sl], ti_v)
    pltpu.sync_copy(ridx_hbm.at[sl], ri_v)
    pltpu.async_copy(x_hbm.at[hi_v], hb_v, sem).wait()
    pltpu.async_copy(x_hbm.at[ti_v], tb_v, sem).wait()
    pltpu.async_copy(rel_hbm.at[ri_v], ph_v, sem).wait()
    pltpu.sync_copy(hb_v, hrows_out.at[sl])
    pltpu.sync_copy(tb_v, trows_out.at[sl])
    pltpu.sync_copy(ph_v, ph_out.at[sl])
    return carry

  lax.fori_loop(0, TITERS, body, 0)


def _tc_score(hrows, trows, ph):
  RB = 512

  def body(h_ref, t_ref, p_ref, o_ref):
    h = h_ref[...]
    t = t_ref[...]
    p = p_ref[...]
    r_re = p[:, :H]
    r_im = p[:, H:]
    h_re = h[:, :H]
    h_im = h[:, H:]
    s_re = h_re * r_re - h_im * r_im - t[:, :H]
    s_im = h_re * r_im + h_im * r_re - t[:, H:]
    dist = jnp.sqrt(s_re * s_re + s_im * s_im).sum(axis=1, keepdims=True)
    o_ref[...] = MARGIN - dist

  return pl.pallas_call(
      body,
      grid=(T // RB,),
      in_specs=[
          pl.BlockSpec((RB, D), lambda i: (i, 0)),
          pl.BlockSpec((RB, D), lambda i: (i, 0)),
          pl.BlockSpec((RB, D), lambda i: (i, 0)),
      ],
      out_specs=pl.BlockSpec((RB, 1), lambda i: (i, 0)),
      out_shape=jax.ShapeDtypeStruct((T, 1), jnp.float32),
  )(hrows, trows, ph)


_sc_agg_cnt = _make_sc_aggregate(with_cnt=True)
_sc_agg_nocnt = _make_sc_aggregate(with_cnt=False)


def kernel(node_emb, rel_emb, basis1, comp1, root1, bias1,
           basis2, comp2, root2, bias2,
           edge_index, edge_type, h_idx, r_idx, t_idx):
  src = edge_index[0].astype(jnp.int32)
  dst = edge_index[1].astype(jnp.int32)
  typ = edge_type.astype(jnp.int32)
  npad_e = E_PAD - E
  pad_ar = jnp.arange(npad_e, dtype=jnp.int32)
  pad_dst = N_NODES + pad_ar % (NPAD - N_NODES)
  pad_src = pad_ar % N_NODES
  src2d = jnp.concatenate([src, pad_src]).reshape(NROWS, CB)
  dst1d = jnp.concatenate([dst, pad_dst])
  typ2d = jnp.concatenate(
      [typ, jnp.zeros((npad_e,), jnp.int32)]).reshape(NROWS, CB)

  idx1d, cs_tbl = _tc_edge_indices(src2d, typ2d, rel_emb)
  idx1d = idx1d.reshape(2 * E_PAD)
  zrows = jnp.zeros((NPAD, D), jnp.float32)
  zcnt = jnp.zeros((NPAD,), jnp.float32)
  x0 = jnp.zeros((NPAD, D), jnp.float32).at[:N_NODES].set(node_emb)

  scaled1 = _tc_build_scaled(
      comp1.T.reshape(2 * NUM_RELS).astype(jnp.float32), x0
  ).reshape(2 * TBL_HALF, D)
  acc, cnt = _sc_agg_cnt(scaled1, idx1d, dst1d, zrows, zcnt)
  cnt2d = cnt.reshape(NPAD, 1)
  x1, scaled2 = _tc_combine(
      acc[0], acc[1], x0, cnt2d, basis1[0], basis1[1],
      root1, bias1.reshape(1, D), relu=True,
      next_comp=comp2.T.reshape(2 * NUM_RELS).astype(jnp.float32))

  acc2, _ = _sc_agg_nocnt(scaled2.reshape(2 * TBL_HALF, D),
                          idx1d, dst1d, zrows, zcnt)
  x2 = _tc_combine(acc2[0], acc2[1], x1, cnt2d, basis2[0], basis2[1],
                   root2, bias2.reshape(1, D), relu=False)

  hrows, trows, ph = _sc_triple_gather(
      x2, cs_tbl, h_idx.astype(jnp.int32), t_idx.astype(jnp.int32),
      r_idx.astype(jnp.int32))
  return _tc_score(hrows, trows, ph).reshape(T)


# speedup vs baseline: 3.6097x; 1.0564x over previous
import functools

import jax
import jax.numpy as jnp
from jax import lax
from jax.experimental import pallas as pl
from jax.experimental.pallas import tpu as pltpu
from jax.experimental.pallas import tpu_sc as plsc

N_NODES = 10000
NPAD = 10240
NUM_RELS = 16
D = 128
H = 64
E = 320000
T = 16384
MARGIN = 9.0

NUM_TILES = 16
CB = 80
NCHUNK = 252
E_PAD = NUM_TILES * NCHUNK * CB
NROWS = E_PAD // CB
E_PER_TILE = NCHUNK * CB
ROWS_PER_TILE = NPAD // NUM_TILES
TBL_HALF = NUM_RELS * NPAD

_MESH = plsc.VectorSubcoreMesh(core_axis_name="c", subcore_axis_name="s")


def _tc_build_scaled(comp_flat, x):
  RB = 512

  def body(comp_ref, x_ref, o_ref):
    xb = x_ref[...]
    for kk in range(2 * NUM_RELS):
      o_ref[kk] = xb * comp_ref[kk]

  return pl.pallas_call(
      body,
      grid=(NPAD // RB,),
      in_specs=[
          pl.BlockSpec(memory_space=pltpu.SMEM),
          pl.BlockSpec((RB, D), lambda i: (i, 0)),
      ],
      out_specs=pl.BlockSpec((2 * NUM_RELS, RB, D), lambda i: (0, i, 0)),
      out_shape=jax.ShapeDtypeStruct((2 * NUM_RELS, NPAD, D), jnp.float32),
  )(comp_flat, x)


def _tc_edge_indices(src2d, typ2d, rel_emb):
  def body(src_ref, typ_ref, rel_ref, o_ref, cs_ref):
    base = typ_ref[...] * NPAD + src_ref[...]
    o_ref[0] = base
    o_ref[1] = base + TBL_HALF
    ph = rel_ref[...]
    cs_ref[...] = jnp.concatenate([jnp.cos(ph), jnp.sin(ph)], axis=1)

  return pl.pallas_call(
      body,
      out_shape=[jax.ShapeDtypeStruct((2, NROWS, CB), jnp.int32),
                 jax.ShapeDtypeStruct((NUM_RELS, D), jnp.float32)],
  )(src2d, typ2d, rel_emb)


def _make_sc_aggregate(with_cnt):
  @functools.partial(
      pl.kernel, mesh=_MESH,
      out_type=[jax.ShapeDtypeStruct((2, NPAD, D), jnp.float32),
                jax.ShapeDtypeStruct((NPAD,), jnp.float32)],
      scratch_types=[
          pltpu.VMEM((CB,), jnp.int32),
          pltpu.VMEM((CB,), jnp.int32),
          pltpu.VMEM((CB,), jnp.int32),
          pltpu.VMEM((CB,), jnp.int32),
          pltpu.VMEM((CB,), jnp.int32),
          pltpu.VMEM((CB,), jnp.int32),
          pltpu.VMEM((CB,), jnp.int32),
          pltpu.VMEM((CB,), jnp.int32),
          pltpu.VMEM((CB, D), jnp.float32),
          pltpu.VMEM((CB, D), jnp.float32),
          pltpu.VMEM((CB, D), jnp.float32),
          pltpu.VMEM((CB, D), jnp.float32),
          pltpu.VMEM((CB,), jnp.float32),
          pltpu.VMEM_SHARED((NPAD, D), jnp.float32),
          pltpu.VMEM_SHARED((NPAD,), jnp.float32),
          pltpu.SemaphoreType.DMA,
          pltpu.SemaphoreType.DMA,
          pltpu.SemaphoreType.DMA,
          pltpu.SemaphoreType.DMA,
          pltpu.SemaphoreType.DMA,
          pltpu.SemaphoreType.DMA,
          pltpu.SemaphoreType.DMA,
      ],
  )
  def k(tbl_hbm, idx_hbm, dst_hbm, zrows_hbm, zcnt_hbm,
        acc_out, cnt_out,
        idxs0, idxs1, idxs2, idxs3, dstc0, dstc1, dstc2, dstc3,
        rows0, rows1, rows2, rows3, ones_v, acc_sh, cnt_sh,
        gsem0, gsem1, sem2, ssem0, ssem1, csem0, csem1):
    c = lax.axis_index("c")
    s = lax.axis_index("s")
    rsl = pl.ds(s * ROWS_PER_TILE, ROWS_PER_TILE)
    pltpu.sync_copy(zrows_hbm.at[rsl], acc_sh.at[rsl])
    if with_cnt:
      pltpu.sync_copy(zcnt_hbm.at[rsl], cnt_sh.at[rsl])
    for j in range(CB // 16):
      ones_v[pl.ds(j * 16, 16)] = jnp.ones((16,), jnp.float32)
    plsc.subcore_barrier()

    ebase = s * E_PER_TILE
    ibase = c * E_PAD + ebase
    rows = [rows0, rows1, rows2, rows3]
    dstc = [dstc0, dstc1, dstc2, dstc3]
    idxs = [idxs0, idxs1, idxs2, idxs3]
    gsem = [gsem0, gsem1]
    ssem = [ssem0, ssem1]
    csem = [csem0, csem1]

    def idxdma(chunk, j):
      pltpu.async_copy(idx_hbm.at[pl.ds(ibase + chunk * CB, CB)],
                       idxs[j], sem2)

    def dstdma(chunk, j):
      pltpu.async_copy(dst_hbm.at[pl.ds(ebase + chunk * CB, CB)],
                       dstc[j], sem2)

    def fdrain(n):
      for _ in range(n):
        pltpu.make_async_copy(dst_hbm.at[pl.ds(0, CB)], dstc0, sem2).wait()

    def gstart(j):
      pltpu.async_copy(tbl_hbm.at[idxs[j]], rows[j], gsem[j % 2])

    def gwait(j):
      pltpu.make_async_copy(tbl_hbm.at[idxs[0]], rows[j],
                            gsem[j % 2]).wait()

    def sstart(j):
      pltpu.async_copy(rows[j], acc_sh.at[dstc[j]], ssem[j % 2], add=True)
      if with_cnt:
        @pl.when(c == 0)
        def _():
          pltpu.async_copy(ones_v, cnt_sh.at[dstc[j]], csem[j % 2], add=True)

    def swait(j):
      pltpu.make_async_copy(rows0, acc_sh.at[dstc0], ssem[j % 2]).wait()
      if with_cnt:
        @pl.when(c == 0)
        def _():
          pltpu.make_async_copy(ones_v, cnt_sh.at[dstc0], csem[j % 2]).wait()

    def slot(q, j, do_swait=True, drain_n=2, g2=True, d2=True, i3=True):
      if do_swait:
        swait(j)
      fdrain(drain_n)
      gwait(j)
      sstart(j)
      if g2:
        gstart((j + 2) % 4)
      if d2:
        dstdma(q + 2, (j + 2) % 4)
      if i3:
        idxdma(q + 3, (j + 3) % 4)

    dstdma(0, 0)
    idxdma(0, 0)
    dstdma(1, 1)
    idxdma(1, 1)
    fdrain(4)
    gstart(0)
    gstart(1)
    idxdma(2, 2)
    slot(0, 0, do_swait=False, drain_n=1)
    slot(1, 1, do_swait=False)
    slot(2, 2)
    slot(3, 3)

    def body(it, carry):
      q0 = it * 4
      slot(q0 + 0, 0)
      slot(q0 + 1, 1)
      slot(q0 + 2, 2)
      slot(q0 + 3, 3)
      return carry

    lax.fori_loop(1, NCHUNK // 4 - 1, body, 0)
    q0 = NCHUNK - 4
    slot(q0 + 0, 0)
    slot(q0 + 1, 1, i3=False)
    slot(q0 + 2, 2, drain_n=1, g2=False, d2=False, i3=False)
    slot(q0 + 3, 3, drain_n=0, g2=False, d2=False, i3=False)
    swait(0)
    swait(1)

    plsc.subcore_barrier()
    pltpu.sync_copy(acc_sh.at[rsl], acc_out.at[c, rsl])
    if with_cnt:
      @pl.when(c == 0)
      def _():
        pltpu.sync_copy(cnt_sh.at[rsl], cnt_out.at[rsl])

  return k


def _tc_combine(acc0, acc1, x, cnt2d, b0, b1, root, bias2d, relu,
                next_comp=None):
  RB = 512

  def compute(a0_ref, a1_ref, x_ref, cnt_ref, b0_ref, b1_ref, root_ref,
              bias_ref):
    msg = jnp.dot(a0_ref[...], b0_ref[...], preferred_element_type=jnp.float32)
    msg = msg + jnp.dot(a1_ref[...], b1_ref[...],
                        preferred_element_type=jnp.float32)
    denom = jnp.maximum(cnt_ref[...], 1.0)
    o = msg / denom + jnp.dot(x_ref[...], root_ref[...],
                              preferred_element_type=jnp.float32)
    o = o + bias_ref[...]
    if relu:
      o = jnp.maximum(o, 0.0)
    return o

  row_specs = [
      pl.BlockSpec((RB, D), lambda i: (i, 0)),
      pl.BlockSpec((RB, D), lambda i: (i, 0)),
      pl.BlockSpec((RB, D), lambda i: (i, 0)),
      pl.BlockSpec((RB, 1), lambda i: (i, 0)),
      pl.BlockSpec((D, D), lambda i: (0, 0)),
      pl.BlockSpec((D, D), lambda i: (0, 0)),
      pl.BlockSpec((D, D), lambda i: (0, 0)),
      pl.BlockSpec((1, D), lambda i: (0, 0)),
  ]

  if next_comp is None:
    def body(a0_ref, a1_ref, x_ref, cnt_ref, b0_ref, b1_ref, root_ref,
             bias_ref, o_ref):
      o_ref[...] = compute(a0_ref, a1_ref, x_ref, cnt_ref, b0_ref, b1_ref,
                           root_ref, bias_ref)

    return pl.pallas_call(
        body,
        grid=(NPAD // RB,),
        in_specs=row_specs,
        out_specs=pl.BlockSpec((RB, D), lambda i: (i, 0)),
        out_shape=jax.ShapeDtypeStruct((NPAD, D), jnp.float32),
    )(acc0, acc1, x, cnt2d, b0, b1, root, bias2d)

  def body2(comp_ref, a0_ref, a1_ref, x_ref, cnt_ref, b0_ref, b1_ref,
            root_ref, bias_ref, o_ref, t_ref):
    o = compute(a0_ref, a1_ref, x_ref, cnt_ref, b0_ref, b1_ref,
                root_ref, bias_ref)
    o_ref[...] = o
    for kk in range(2 * NUM_RELS):
      t_ref[kk] = o * comp_ref[kk]

  return pl.pallas_call(
      body2,
      grid=(NPAD // RB,),
      in_specs=[pl.BlockSpec(memory_space=pltpu.SMEM)] + row_specs,
      out_specs=[
          pl.BlockSpec((RB, D), lambda i: (i, 0)),
          pl.BlockSpec((2 * NUM_RELS, RB, D), lambda i: (0, i, 0)),
      ],
      out_shape=[
          jax.ShapeDtypeStruct((NPAD, D), jnp.float32),
          jax.ShapeDtypeStruct((2 * NUM_RELS, NPAD, D), jnp.float32),
      ],
  )(next_comp, acc0, acc1, x, cnt2d, b0, b1, root, bias2d)


NW = 32
T_PER_W = T // NW
TCHUNK = 128
TITERS = T_PER_W // TCHUNK


@functools.partial(
    pl.kernel, mesh=_MESH,
    out_type=[jax.ShapeDtypeStruct((T, D), jnp.float32),
              jax.ShapeDtypeStruct((T, D), jnp.float32),
              jax.ShapeDtypeStruct((T, D), jnp.float32)],
    scratch_types=[
        pltpu.VMEM((TCHUNK,), jnp.int32),
        pltpu.VMEM((TCHUNK,), jnp.int32),
        pltpu.VMEM((TCHUNK,), jnp.int32),
        pltpu.VMEM((2, TCHUNK, D), jnp.float32),
        pltpu.VMEM((2, TCHUNK, D), jnp.float32),
        pltpu.VMEM((2, TCHUNK, D), jnp.float32),
        pltpu.SemaphoreType.DMA,
        pltpu.SemaphoreType.DMA,
        pltpu.SemaphoreType.DMA,
    ],
)
def _sc_triple_gather(x_hbm, rel_hbm, hidx_hbm, tidx_hbm, ridx_hbm,
                      hrows_out, trows_out, ph_out,
                      hi_v, ti_v, ri_v, hb_v, tb_v, ph_v,
                      gsemh, gsemt, gsemr):
  c = lax.axis_index("c")
  s = lax.axis_index("s")
  wid = s * 2 + c
  base = wid * T_PER_W

  def fetch_idx(it):
    sl = pl.ds(base + it * TCHUNK, TCHUNK)
    pltpu.sync_copy(hidx_hbm.at[sl], hi_v)
    pltpu.sync_copy(tidx_hbm.at[sl], ti_v)
    pltpu.sync_copy(ridx_hbm.at[sl], ri_v)

  def gstart(b):
    pltpu.async_copy(x_hbm.at[hi_v], hb_v.at[b], gsemh)
    pltpu.async_copy(x_hbm.at[ti_v], tb_v.at[b], gsemt)
    pltpu.async_copy(rel_hbm.at[ri_v], ph_v.at[b], gsemr)

  def gwait(b):
    pltpu.make_async_copy(x_hbm.at[hi_v], hb_v.at[b], gsemh).wait()
    pltpu.make_async_copy(x_hbm.at[ti_v], tb_v.at[b], gsemt).wait()
    pltpu.make_async_copy(rel_hbm.at[ri_v], ph_v.at[b], gsemr).wait()

  def put(it, b):
    sl = pl.ds(base + it * TCHUNK, TCHUNK)
    pltpu.sync_copy(hb_v.at[b], hrows_out.at[sl])
    pltpu.sync_copy(tb_v.at[b], trows_out.at[sl])
    pltpu.sync_copy(ph_v.at[b], ph_out.at[sl])

  fetch_idx(0)
  gstart(0)
  for it in range(TITERS):
    b = it % 2
    gwait(b)
    if it + 1 < TITERS:
      fetch_idx(it + 1)
      gstart(1 - b)
    put(it, b)


def _tc_score(hrows, trows, ph):
  RB = 512

  def body(h_ref, t_ref, p_ref, o_ref):
    h = h_ref[...]
    t = t_ref[...]
    p = p_ref[...]
    r_re = p[:, :H]
    r_im = p[:, H:]
    h_re = h[:, :H]
    h_im = h[:, H:]
    s_re = h_re * r_re - h_im * r_im - t[:, :H]
    s_im = h_re * r_im + h_im * r_re - t[:, H:]
    dist = jnp.sqrt(s_re * s_re + s_im * s_im).sum(axis=1, keepdims=True)
    o_ref[...] = MARGIN - dist

  return pl.pallas_call(
      body,
      grid=(T // RB,),
      in_specs=[
          pl.BlockSpec((RB, D), lambda i: (i, 0)),
          pl.BlockSpec((RB, D), lambda i: (i, 0)),
          pl.BlockSpec((RB, D), lambda i: (i, 0)),
      ],
      out_specs=pl.BlockSpec((RB, 1), lambda i: (i, 0)),
      out_shape=jax.ShapeDtypeStruct((T, 1), jnp.float32),
  )(hrows, trows, ph)


_sc_agg_cnt = _make_sc_aggregate(with_cnt=True)
_sc_agg_nocnt = _make_sc_aggregate(with_cnt=False)


def kernel(node_emb, rel_emb, basis1, comp1, root1, bias1,
           basis2, comp2, root2, bias2,
           edge_index, edge_type, h_idx, r_idx, t_idx):
  src = edge_index[0].astype(jnp.int32)
  dst = edge_index[1].astype(jnp.int32)
  typ = edge_type.astype(jnp.int32)
  npad_e = E_PAD - E
  pad_ar = jnp.arange(npad_e, dtype=jnp.int32)
  pad_dst = N_NODES + pad_ar % (NPAD - N_NODES)
  pad_src = pad_ar % N_NODES
  src2d = jnp.concatenate([src, pad_src]).reshape(NROWS, CB)
  dst1d = jnp.concatenate([dst, pad_dst])
  typ2d = jnp.concatenate(
      [typ, jnp.zeros((npad_e,), jnp.int32)]).reshape(NROWS, CB)

  idx1d, cs_tbl = _tc_edge_indices(src2d, typ2d, rel_emb)
  idx1d = idx1d.reshape(2 * E_PAD)
  zrows = jnp.zeros((NPAD, D), jnp.float32)
  zcnt = jnp.zeros((NPAD,), jnp.float32)
  x0 = jnp.zeros((NPAD, D), jnp.float32).at[:N_NODES].set(node_emb)

  scaled1 = _tc_build_scaled(
      comp1.T.reshape(2 * NUM_RELS).astype(jnp.float32), x0
  ).reshape(2 * TBL_HALF, D)
  acc, cnt = _sc_agg_cnt(scaled1, idx1d, dst1d, zrows, zcnt)
  cnt2d = cnt.reshape(NPAD, 1)
  x1, scaled2 = _tc_combine(
      acc[0], acc[1], x0, cnt2d, basis1[0], basis1[1],
      root1, bias1.reshape(1, D), relu=True,
      next_comp=comp2.T.reshape(2 * NUM_RELS).astype(jnp.float32))

  acc2, _ = _sc_agg_nocnt(scaled2.reshape(2 * TBL_HALF, D),
                          idx1d, dst1d, zrows, zcnt)
  x2 = _tc_combine(acc2[0], acc2[1], x1, cnt2d, basis2[0], basis2[1],
                   root2, bias2.reshape(1, D), relu=False)

  hrows, trows, ph = _sc_triple_gather(
      x2, cs_tbl, h_idx.astype(jnp.int32), t_idx.astype(jnp.int32),
      r_idx.astype(jnp.int32))
  return _tc_score(hrows, trows, ph).reshape(T)


# speedup vs baseline: 3.8806x; 1.0751x over previous
import functools

import jax
import jax.numpy as jnp
from jax import lax
from jax.experimental import pallas as pl
from jax.experimental.pallas import tpu as pltpu
from jax.experimental.pallas import tpu_sc as plsc

N_NODES = 10000
NPAD = 10240
NUM_RELS = 16
D = 128
H = 64
E = 320000
T = 16384
MARGIN = 9.0

NUM_TILES = 16
CB = 80
NCHUNK = 252
E_PAD = NUM_TILES * NCHUNK * CB
NROWS = E_PAD // CB
E_PER_TILE = NCHUNK * CB
ROWS_PER_TILE = NPAD // NUM_TILES
TBL_HALF = NUM_RELS * NPAD

_MESH = plsc.VectorSubcoreMesh(core_axis_name="c", subcore_axis_name="s")


def _tc_build_scaled(comp_flat, x):
  RB = 512

  def body(comp_ref, x_ref, o_ref):
    xb = x_ref[...]
    for kk in range(2 * NUM_RELS):
      o_ref[kk] = xb * comp_ref[kk]

  return pl.pallas_call(
      body,
      grid=(NPAD // RB,),
      in_specs=[
          pl.BlockSpec(memory_space=pltpu.SMEM),
          pl.BlockSpec((RB, D), lambda i: (i, 0)),
      ],
      out_specs=pl.BlockSpec((2 * NUM_RELS, RB, D), lambda i: (0, i, 0)),
      out_shape=jax.ShapeDtypeStruct((2 * NUM_RELS, NPAD, D), jnp.float32),
  )(comp_flat, x)


def _tc_edge_indices(src2d, typ2d, rel_emb):
  def body(src_ref, typ_ref, rel_ref, o_ref, cs_ref):
    base = typ_ref[...] * NPAD + src_ref[...]
    o_ref[0] = base
    o_ref[1] = base + TBL_HALF
    ph = rel_ref[...]
    cs_ref[...] = jnp.concatenate([jnp.cos(ph), jnp.sin(ph)], axis=1)

  return pl.pallas_call(
      body,
      out_shape=[jax.ShapeDtypeStruct((2, NROWS, CB), jnp.int32),
                 jax.ShapeDtypeStruct((NUM_RELS, D), jnp.float32)],
  )(src2d, typ2d, rel_emb)


def _make_sc_aggregate(with_cnt):
  @functools.partial(
      pl.kernel, mesh=_MESH,
      out_type=[jax.ShapeDtypeStruct((2, NPAD, D), jnp.float32),
                jax.ShapeDtypeStruct((NPAD,), jnp.float32)],
      scratch_types=[
          pltpu.VMEM((CB,), jnp.int32),
          pltpu.VMEM((CB,), jnp.int32),
          pltpu.VMEM((CB,), jnp.int32),
          pltpu.VMEM((CB,), jnp.int32),
          pltpu.VMEM((CB,), jnp.int32),
          pltpu.VMEM((CB,), jnp.int32),
          pltpu.VMEM((CB,), jnp.int32),
          pltpu.VMEM((CB,), jnp.int32),
          pltpu.VMEM((CB, D), jnp.float32),
          pltpu.VMEM((CB, D), jnp.float32),
          pltpu.VMEM((CB, D), jnp.float32),
          pltpu.VMEM((CB, D), jnp.float32),
          pltpu.VMEM((CB,), jnp.float32),
          pltpu.VMEM_SHARED((NPAD, D), jnp.float32),
          pltpu.VMEM_SHARED((NPAD,), jnp.float32),
          pltpu.SemaphoreType.DMA,
          pltpu.SemaphoreType.DMA,
          pltpu.SemaphoreType.DMA,
          pltpu.SemaphoreType.DMA,
          pltpu.SemaphoreType.DMA,
          pltpu.SemaphoreType.DMA,
          pltpu.SemaphoreType.DMA,
      ],
  )
  def k(tbl_hbm, idx_hbm, dst_hbm, zrows_hbm, zcnt_hbm,
        acc_out, cnt_out,
        idxs0, idxs1, idxs2, idxs3, dstc0, dstc1, dstc2, dstc3,
        rows0, rows1, rows2, rows3, ones_v, acc_sh, cnt_sh,
        gsem0, gsem1, sem2, ssem0, ssem1, csem0, csem1):
    c = lax.axis_index("c")
    s = lax.axis_index("s")
    rsl = pl.ds(s * ROWS_PER_TILE, ROWS_PER_TILE)
    pltpu.sync_copy(zrows_hbm.at[rsl], acc_sh.at[rsl])
    if with_cnt:
      pltpu.sync_copy(zcnt_hbm.at[rsl], cnt_sh.at[rsl])
    for j in range(CB // 16):
      ones_v[pl.ds(j * 16, 16)] = jnp.ones((16,), jnp.float32)
    plsc.subcore_barrier()

    ebase = s * E_PER_TILE
    ibase = c * E_PAD + ebase
    rows = [rows0, rows1, rows2, rows3]
    dstc = [dstc0, dstc1, dstc2, dstc3]
    idxs = [idxs0, idxs1, idxs2, idxs3]
    gsem = [gsem0, gsem1]
    ssem = [ssem0, ssem1]
    csem = [csem0, csem1]

    def idxdma(chunk, j):
      pltpu.async_copy(idx_hbm.at[pl.ds(ibase + chunk * CB, CB)],
                       idxs[j], sem2)

    def dstdma(chunk, j):
      pltpu.async_copy(dst_hbm.at[pl.ds(ebase + chunk * CB, CB)],
                       dstc[j], sem2)

    def fdrain(n):
      for _ in range(n):
        pltpu.make_async_copy(dst_hbm.at[pl.ds(0, CB)], dstc0, sem2).wait()

    def gstart(j):
      pltpu.async_copy(tbl_hbm.at[idxs[j]], rows[j], gsem[j % 2])

    def gwait(j):
      pltpu.make_async_copy(tbl_hbm.at[idxs[0]], rows[j],
                            gsem[j % 2]).wait()

    def sstart(j):
      pltpu.async_copy(rows[j], acc_sh.at[dstc[j]], ssem[j % 2], add=True)
      if with_cnt:
        @pl.when(c == 0)
        def _():
          pltpu.async_copy(ones_v, cnt_sh.at[dstc[j]], csem[j % 2], add=True)

    def swait(j):
      pltpu.make_async_copy(rows0, acc_sh.at[dstc0], ssem[j % 2]).wait()
      if with_cnt:
        @pl.when(c == 0)
        def _():
          pltpu.make_async_copy(ones_v, cnt_sh.at[dstc0], csem[j % 2]).wait()

    def slot(q, j, do_swait=True, drain_n=2, g2=True, d2=True, i3=True):
      if do_swait:
        swait(j)
      fdrain(drain_n)
      gwait(j)
      sstart(j)
      if g2:
        gstart((j + 2) % 4)
      if d2:
        dstdma(q + 2, (j + 2) % 4)
      if i3:
        idxdma(q + 3, (j + 3) % 4)

    dstdma(0, 0)
    idxdma(0, 0)
    dstdma(1, 1)
    idxdma(1, 1)
    fdrain(4)
    gstart(0)
    gstart(1)
    idxdma(2, 2)
    slot(0, 0, do_swait=False, drain_n=1)
    slot(1, 1, do_swait=False)
    slot(2, 2)
    slot(3, 3)

    def body(it, carry):
      q0 = it * 4
      slot(q0 + 0, 0)
      slot(q0 + 1, 1)
      slot(q0 + 2, 2)
      slot(q0 + 3, 3)
      return carry

    lax.fori_loop(1, NCHUNK // 4 - 1, body, 0)
    q0 = NCHUNK - 4
    slot(q0 + 0, 0)
    slot(q0 + 1, 1, i3=False)
    slot(q0 + 2, 2, drain_n=1, g2=False, d2=False, i3=False)
    slot(q0 + 3, 3, drain_n=0, g2=False, d2=False, i3=False)
    swait(0)
    swait(1)

    plsc.subcore_barrier()
    pltpu.sync_copy(acc_sh.at[rsl], acc_out.at[c, rsl])
    if with_cnt:
      @pl.when(c == 0)
      def _():
        pltpu.sync_copy(cnt_sh.at[rsl], cnt_out.at[rsl])

  return k


def _tc_combine(acc0, acc1, x, cnt2d, b0, b1, root, bias2d, relu,
                next_comp=None):
  RB = 512

  def compute(a0_ref, a1_ref, x_ref, cnt_ref, b0_ref, b1_ref, root_ref,
              bias_ref):
    msg = jnp.dot(a0_ref[...], b0_ref[...], preferred_element_type=jnp.float32)
    msg = msg + jnp.dot(a1_ref[...], b1_ref[...],
                        preferred_element_type=jnp.float32)
    denom = jnp.maximum(cnt_ref[...], 1.0)
    o = msg / denom + jnp.dot(x_ref[...], root_ref[...],
                              preferred_element_type=jnp.float32)
    o = o + bias_ref[...]
    if relu:
      o = jnp.maximum(o, 0.0)
    return o

  row_specs = [
      pl.BlockSpec((RB, D), lambda i: (i, 0)),
      pl.BlockSpec((RB, D), lambda i: (i, 0)),
      pl.BlockSpec((RB, D), lambda i: (i, 0)),
      pl.BlockSpec((RB, 1), lambda i: (i, 0)),
      pl.BlockSpec((D, D), lambda i: (0, 0)),
      pl.BlockSpec((D, D), lambda i: (0, 0)),
      pl.BlockSpec((D, D), lambda i: (0, 0)),
      pl.BlockSpec((1, D), lambda i: (0, 0)),
  ]

  if next_comp is None:
    def body(a0_ref, a1_ref, x_ref, cnt_ref, b0_ref, b1_ref, root_ref,
             bias_ref, o_ref):
      o_ref[...] = compute(a0_ref, a1_ref, x_ref, cnt_ref, b0_ref, b1_ref,
                           root_ref, bias_ref)

    return pl.pallas_call(
        body,
        grid=(NPAD // RB,),
        in_specs=row_specs,
        out_specs=pl.BlockSpec((RB, D), lambda i: (i, 0)),
        out_shape=jax.ShapeDtypeStruct((NPAD, D), jnp.float32),
    )(acc0, acc1, x, cnt2d, b0, b1, root, bias2d)

  def body2(comp_ref, a0_ref, a1_ref, x_ref, cnt_ref, b0_ref, b1_ref,
            root_ref, bias_ref, o_ref, t_ref):
    o = compute(a0_ref, a1_ref, x_ref, cnt_ref, b0_ref, b1_ref,
                root_ref, bias_ref)
    o_ref[...] = o
    for kk in range(2 * NUM_RELS):
      t_ref[kk] = o * comp_ref[kk]

  return pl.pallas_call(
      body2,
      grid=(NPAD // RB,),
      in_specs=[pl.BlockSpec(memory_space=pltpu.SMEM)] + row_specs,
      out_specs=[
          pl.BlockSpec((RB, D), lambda i: (i, 0)),
          pl.BlockSpec((2 * NUM_RELS, RB, D), lambda i: (0, i, 0)),
      ],
      out_shape=[
          jax.ShapeDtypeStruct((NPAD, D), jnp.float32),
          jax.ShapeDtypeStruct((2 * NUM_RELS, NPAD, D), jnp.float32),
      ],
  )(next_comp, acc0, acc1, x, cnt2d, b0, b1, root, bias2d)


NW = 32
T_PER_W = T // NW
TCHUNK = 128
TITERS = T_PER_W // TCHUNK


@functools.partial(
    pl.kernel, mesh=_MESH,
    out_type=[jax.ShapeDtypeStruct((T, D), jnp.float32),
              jax.ShapeDtypeStruct((T, D), jnp.float32)],
    scratch_types=[
        pltpu.VMEM((TCHUNK,), jnp.int32),
        pltpu.VMEM((TCHUNK,), jnp.int32),
        pltpu.VMEM((2, TCHUNK, D), jnp.float32),
        pltpu.VMEM((2, TCHUNK, D), jnp.float32),
        pltpu.SemaphoreType.DMA,
        pltpu.SemaphoreType.DMA,
    ],
)
def _sc_triple_gather(x_hbm, hidx_hbm, tidx_hbm,
                      hrows_out, trows_out,
                      hi_v, ti_v, hb_v, tb_v, gsemh, gsemt):
  c = lax.axis_index("c")
  s = lax.axis_index("s")
  wid = s * 2 + c
  base = wid * T_PER_W

  def fetch_idx(it):
    sl = pl.ds(base + it * TCHUNK, TCHUNK)
    pltpu.sync_copy(hidx_hbm.at[sl], hi_v)
    pltpu.sync_copy(tidx_hbm.at[sl], ti_v)

  def gstart(b):
    pltpu.async_copy(x_hbm.at[hi_v], hb_v.at[b], gsemh)
    pltpu.async_copy(x_hbm.at[ti_v], tb_v.at[b], gsemt)

  def gwait(b):
    pltpu.make_async_copy(x_hbm.at[hi_v], hb_v.at[b], gsemh).wait()
    pltpu.make_async_copy(x_hbm.at[ti_v], tb_v.at[b], gsemt).wait()

  def put(it, b):
    sl = pl.ds(base + it * TCHUNK, TCHUNK)
    pltpu.sync_copy(hb_v.at[b], hrows_out.at[sl])
    pltpu.sync_copy(tb_v.at[b], trows_out.at[sl])

  fetch_idx(0)
  gstart(0)
  for it in range(TITERS):
    b = it % 2
    gwait(b)
    if it + 1 < TITERS:
      fetch_idx(it + 1)
      gstart(1 - b)
    put(it, b)


def _tc_score(hrows, trows, ridx2d, cs_tbl):
  RB = 512

  def body(h_ref, t_ref, r_ref, cs_ref, o_ref):
    h = h_ref[...]
    t = t_ref[...]
    rid = r_ref[...]
    p = jnp.zeros((RB, D), jnp.float32)
    for kk in range(NUM_RELS):
      p = p + jnp.where(rid == kk, 1.0, 0.0) * cs_ref[kk][None, :]
    r_re = p[:, :H]
    r_im = p[:, H:]
    h_re = h[:, :H]
    h_im = h[:, H:]
    s_re = h_re * r_re - h_im * r_im - t[:, :H]
    s_im = h_re * r_im + h_im * r_re - t[:, H:]
    dist = jnp.sqrt(s_re * s_re + s_im * s_im).sum(axis=1, keepdims=True)
    o_ref[...] = MARGIN - dist

  return pl.pallas_call(
      body,
      grid=(T // RB,),
      in_specs=[
          pl.BlockSpec((RB, D), lambda i: (i, 0)),
          pl.BlockSpec((RB, D), lambda i: (i, 0)),
          pl.BlockSpec((RB, 1), lambda i: (i, 0)),
          pl.BlockSpec((NUM_RELS, D), lambda i: (0, 0)),
      ],
      out_specs=pl.BlockSpec((RB, 1), lambda i: (i, 0)),
      out_shape=jax.ShapeDtypeStruct((T, 1), jnp.float32),
  )(hrows, trows, ridx2d, cs_tbl)


_sc_agg_cnt = _make_sc_aggregate(with_cnt=True)
_sc_agg_nocnt = _make_sc_aggregate(with_cnt=False)


def kernel(node_emb, rel_emb, basis1, comp1, root1, bias1,
           basis2, comp2, root2, bias2,
           edge_index, edge_type, h_idx, r_idx, t_idx):
  src = edge_index[0].astype(jnp.int32)
  dst = edge_index[1].astype(jnp.int32)
  typ = edge_type.astype(jnp.int32)
  npad_e = E_PAD - E
  pad_ar = jnp.arange(npad_e, dtype=jnp.int32)
  pad_dst = N_NODES + pad_ar % (NPAD - N_NODES)
  pad_src = pad_ar % N_NODES
  src2d = jnp.concatenate([src, pad_src]).reshape(NROWS, CB)
  dst1d = jnp.concatenate([dst, pad_dst])
  typ2d = jnp.concatenate(
      [typ, jnp.zeros((npad_e,), jnp.int32)]).reshape(NROWS, CB)

  idx1d, cs_tbl = _tc_edge_indices(src2d, typ2d, rel_emb)
  idx1d = idx1d.reshape(2 * E_PAD)
  zrows = jnp.zeros((NPAD, D), jnp.float32)
  zcnt = jnp.zeros((NPAD,), jnp.float32)
  x0 = jnp.zeros((NPAD, D), jnp.float32).at[:N_NODES].set(node_emb)

  scaled1 = _tc_build_scaled(
      comp1.T.reshape(2 * NUM_RELS).astype(jnp.float32), x0
  ).reshape(2 * TBL_HALF, D)
  acc, cnt = _sc_agg_cnt(scaled1, idx1d, dst1d, zrows, zcnt)
  cnt2d = cnt.reshape(NPAD, 1)
  x1, scaled2 = _tc_combine(
      acc[0], acc[1], x0, cnt2d, basis1[0], basis1[1],
      root1, bias1.reshape(1, D), relu=True,
      next_comp=comp2.T.reshape(2 * NUM_RELS).astype(jnp.float32))

  acc2, _ = _sc_agg_nocnt(scaled2.reshape(2 * TBL_HALF, D),
                          idx1d, dst1d, zrows, zcnt)
  x2 = _tc_combine(acc2[0], acc2[1], x1, cnt2d, basis2[0], basis2[1],
                   root2, bias2.reshape(1, D), relu=False)

  hrows, trows = _sc_triple_gather(
      x2, h_idx.astype(jnp.int32), t_idx.astype(jnp.int32))
  return _tc_score(hrows, trows, r_idx.astype(jnp.int32).reshape(T, 1),
                   cs_tbl).reshape(T)
